# Initial kernel scaffold; baseline (speedup 1.0000x reference)
import jax
import jax.numpy as jnp
from jax.experimental import pallas as pl


def kernel(x, batch, W, b):
    raise NotImplementedError("write your pallas kernel here")


# speedup vs baseline: 4.2101x; 4.2101x over previous
import functools

import jax
import jax.numpy as jnp
from jax import lax
from jax.experimental import pallas as pl
from jax.experimental.pallas import tpu as pltpu
from jax.experimental.pallas import tpu_sc as plsc

N_NODES = 100000
D = 128
NUM_SEGMENTS = 512
WIN = 128
NWIN = N_NODES // WIN
TAIL = N_NODES - NWIN * WIN


def _make_sc_segment_sum():
    info = plsc.get_sparse_core_info()
    nc, ns = info.num_cores, info.num_subcores
    nwk = nc * ns
    max_rounds = (NWIN + nwk - 1) // nwk
    rows_per_sub = NUM_SEGMENTS // ns

    mesh = plsc.VectorSubcoreMesh(core_axis_name="c", subcore_axis_name="s")

    @functools.partial(
        pl.kernel,
        mesh=mesh,
        out_type=jax.ShapeDtypeStruct((nc, NUM_SEGMENTS, D), jnp.float32),
        scratch_types=[
            pltpu.VMEM((WIN,), jnp.int32),
            pltpu.VMEM((TAIL,), jnp.int32),
            pltpu.VMEM((WIN, D), jnp.float32),
            pltpu.VMEM_SHARED((NUM_SEGMENTS, D), jnp.float32),
        ],
    )
    def seg_sum(x_hbm, idsm_hbm, idst_hbm, zrows_hbm, out_hbm,
                idxbuf, idxtail, xbuf, acc):
        c = lax.axis_index("c")
        s = lax.axis_index("s")
        wid = s * nc + c

        pltpu.sync_copy(zrows_hbm, xbuf.at[pl.ds(0, rows_per_sub)])
        pltpu.sync_copy(xbuf.at[pl.ds(0, rows_per_sub)],
                        acc.at[pl.ds(s * rows_per_sub, rows_per_sub)])
        plsc.subcore_barrier()

        def body(k, carry):
            g = wid + nwk * k

            @pl.when(g < NWIN)
            def _():
                pltpu.sync_copy(idsm_hbm.at[g], idxbuf)
                pltpu.sync_copy(x_hbm.at[pl.ds(g * WIN, WIN)], xbuf)
                pltpu.sync_copy(xbuf, acc.at[idxbuf], add=True)
            return carry

        lax.fori_loop(0, max_rounds, body, 0)

        @pl.when(wid == nwk - 1)
        def _():
            pltpu.sync_copy(idst_hbm, idxtail)
            pltpu.sync_copy(x_hbm.at[pl.ds(NWIN * WIN, TAIL)],
                            xbuf.at[pl.ds(0, TAIL)])
            pltpu.sync_copy(xbuf.at[pl.ds(0, TAIL)], acc.at[idxtail], add=True)

        plsc.subcore_barrier()

        pltpu.sync_copy(acc.at[pl.ds(s * rows_per_sub, rows_per_sub)],
                        xbuf.at[pl.ds(0, rows_per_sub)])
        pltpu.sync_copy(xbuf.at[pl.ds(0, rows_per_sub)],
                        out_hbm.at[c, pl.ds(s * rows_per_sub, rows_per_sub)])

    return seg_sum, rows_per_sub


def _finish_body(p_ref, w_ref, b_ref, o_ref):
    pooled = jnp.sum(p_ref[...], axis=0)
    y = lax.dot_general(pooled, w_ref[...],
                        dimension_numbers=(((1,), (1,)), ((), ())),
                        preferred_element_type=jnp.float32)
    y = y + b_ref[...]
    o_ref[...] = jnp.where(y >= 0.0, y, 0.01 * y)


def kernel(x, batch, W, b):
    ids = batch.astype(jnp.int32)
    ids_main = ids[: NWIN * WIN].reshape(NWIN, WIN)
    ids_tail = ids[NWIN * WIN:]
    seg_sum, rows_per_sub = _make_sc_segment_sum()
    zrows = jnp.zeros((rows_per_sub, D), jnp.float32)
    partials = seg_sum(x, ids_main, ids_tail, zrows)
    out = pl.pallas_call(
        _finish_body,
        out_shape=jax.ShapeDtypeStruct((NUM_SEGMENTS, D), jnp.float32),
    )(partials, W, b.reshape(1, D))
    return out
